# Initial kernel scaffold; baseline (speedup 1.0000x reference)
import jax
import jax.numpy as jnp
from jax.experimental import pallas as pl


def kernel(x, labels):
    raise NotImplementedError("write your pallas kernel here")


# speedup vs baseline: 2.0967x; 2.0967x over previous
import functools

import jax
import jax.numpy as jnp
from jax import lax
from jax.experimental import pallas as pl
from jax.experimental.pallas import tpu as pltpu
from jax.experimental.pallas import tpu_sc as plsc

TOPK = 16
L = 16
NC = 2
NS = 16
NW = NC * NS
SUP = 128

_NEG_INF = float("-inf")


def _topk_row(buf, n, iota):
    n_sup = n // SUP

    def insert_chunk(carry, cbase):
        top_v, top_i, thr = carry
        v = buf[pl.ds(cbase, L)]
        gidx = iota + cbase

        def w_cond(c):
            _, _, th, rem = c
            return jnp.any(rem & (v > th))

        def w_body(c):
            tv, ti, th, rem = c
            cand = rem & (v > th)
            s = jnp.max(jnp.where(cand, v, _NEG_INF))
            is_s = cand & (v == s)
            cidx = jnp.min(jnp.where(is_s, gidx, jnp.int32(2**31 - 1)))
            rem2 = rem & (gidx != cidx)
            mnm = tv == th
            emax = jnp.max(jnp.where(mnm, ti, jnp.int32(-1)))
            evict = mnm & (ti == emax)
            tv2 = jnp.where(evict, s, tv)
            ti2 = jnp.where(evict, cidx, ti)
            th2 = jnp.broadcast_to(jnp.min(tv2), (L,))
            return tv2, ti2, th2, rem2

        rem0 = gidx == gidx
        tv, ti, th, _ = lax.while_loop(w_cond, w_body, (top_v, top_i, thr, rem0))
        return tv, ti, th

    def sup_body(s, carry):
        top_v, top_i, thr = carry
        base = s * SUP
        m = buf[pl.ds(base, L)]
        for j in range(1, SUP // L):
            m = jnp.maximum(m, buf[pl.ds(base + j * L, L)])
        hit = jnp.any(m > thr)

        def do_update(c):
            for j in range(SUP // L):
                c = insert_chunk(c, base + j * L)
            return c

        return lax.cond(hit, do_update, lambda c: c, carry)

    init = (
        jnp.full((L,), _NEG_INF, jnp.float32),
        iota,
        jnp.full((L,), _NEG_INF, jnp.float32),
    )
    top_v, top_i, _ = lax.fori_loop(0, n_sup, sup_body, init)
    return top_v, top_i


def _tie_fix(tv, ti, iota, f32_s, i32_s):
    p_even = lax.bitwise_xor(iota, jnp.int32(1))
    p_odd = jnp.clip(lax.bitwise_xor(iota - 1, jnp.int32(1)) + 1, 0, L - 1)
    for p in (p_even, p_odd, p_even, p_odd):
        f32_s[...] = tv
        i32_s[...] = ti
        pv = plsc.load_gather(f32_s, [p])
        pi = plsc.load_gather(i32_s, [p])
        left = iota < p
        win = (tv > pv) | ((tv == pv) & (ti < pi))
        take_self = (win == left) | (p == iota)
        tv = jnp.where(take_self, tv, pv)
        ti = jnp.where(take_self, ti, pi)
    return tv, ti


def _build_sc_call(b, n):
    rows_per_w = b // NW
    mesh = plsc.VectorSubcoreMesh(core_axis_name="c", subcore_axis_name="s")

    @functools.partial(
        pl.kernel,
        out_type=[
            jax.ShapeDtypeStruct((b * TOPK,), jnp.float32),
            jax.ShapeDtypeStruct((b * TOPK,), jnp.int32),
        ],
        mesh=mesh,
        compiler_params=pltpu.CompilerParams(needs_layout_passes=False),
        scratch_types=[
            pltpu.VMEM((n,), jnp.float32),
            pltpu.VMEM((n,), jnp.float32),
            pltpu.VMEM((TOPK,), jnp.float32),
            pltpu.VMEM((TOPK,), jnp.int32),
            pltpu.VMEM((TOPK,), jnp.int32),
            pltpu.SemaphoreType.DMA,
            pltpu.SemaphoreType.DMA,
            pltpu.SemaphoreType.DMA,
        ],
    )
    def sc_topk(x_hbm, labels_hbm, outv_hbm, outi_hbm,
                buf_a, buf_b, f32_s, i32_s, lbl_s, sem_a, sem_b, sem_g):
        wid = lax.axis_index("s") * NC + lax.axis_index("c")
        base_row = wid * rows_per_w
        iota = lax.iota(jnp.int32, L)

        bufs = (buf_a, buf_b)
        sems = (sem_a, sem_b)
        copies = [None] * rows_per_w
        copies[0] = pltpu.async_copy(x_hbm.at[base_row], buf_a, sem_a)
        for r in range(rows_per_w):
            if r + 1 < rows_per_w:
                copies[r + 1] = pltpu.async_copy(
                    x_hbm.at[base_row + r + 1], bufs[(r + 1) % 2], sems[(r + 1) % 2])
            copies[r].wait()
            top_v, top_i = _topk_row(bufs[r % 2], n, iota)
            top_v, top_i = plsc.sort_key_val(top_v, top_i, descending=True)
            top_v, top_i = _tie_fix(top_v, top_i, iota, f32_s, i32_s)
            i32_s[...] = top_i
            pltpu.async_copy(labels_hbm.at[i32_s], lbl_s, sem_g).wait()
            f32_s[...] = top_v
            out_off = (base_row + r) * TOPK
            pltpu.sync_copy(f32_s, outv_hbm.at[pl.ds(out_off, TOPK)])
            pltpu.sync_copy(lbl_s, outi_hbm.at[pl.ds(out_off, TOPK)])

    return sc_topk


def kernel(x, labels):
    b, n = x.shape
    out_v, out_l = _build_sc_call(b, n)(x, labels)
    return out_v.reshape(b, TOPK), out_l.reshape(b, TOPK)
